# baseline (speedup 1.0000x reference)
import math

import jax
import jax.numpy as jnp
from jax import lax
from jax.experimental import pallas as pl

_H_LIST = [1, 1, 2]
_C = 128
_DEPTH = 3
_RECURRENCE = 3


def _attend(x, adjT, mask, Wall, We, H, first_layer):
    if first_layer:
        proj = x * Wall
    else:
        proj = jnp.dot(x, Wall, preferred_element_type=jnp.float32)
    HC = H * _C
    k = proj[:, 0:HC]
    q = proj[:, HC:2 * HC]
    v = proj[:, 2 * HC:3 * HC]
    skip = proj[:, 3 * HC:]

    scale = 1.0 / math.sqrt(float(_C))
    head_outs = []
    for h in range(H):
        qh = q[:, h * _C:(h + 1) * _C]
        kh = k[:, h * _C:(h + 1) * _C]
        vh = v[:, h * _C:(h + 1) * _C]
        weh = We[0:1, h * _C:(h + 1) * _C]
        S = lax.dot_general(qh, kh, (((1,), (1,)), ((), ())),
                            preferred_element_type=jnp.float32)
        qe = jnp.sum(qh * weh, axis=1, keepdims=True)
        S = S + adjT * qe
        S = jnp.where(mask, S, -jnp.inf)
        m = jnp.max(S, axis=1, keepdims=True)
        m = jnp.where(m == -jnp.inf, 0.0, m)
        P = jnp.exp((S - m) * scale)
        ones_col = jnp.ones((vh.shape[0], 1), jnp.float32)
        full = jnp.dot(P, jnp.concatenate([vh, ones_col], axis=1),
                       preferred_element_type=jnp.float32)
        pv = full[:, :_C]
        denom = full[:, _C:_C + 1]
        pe = jnp.dot(P * adjT, ones_col,
                     preferred_element_type=jnp.float32)
        r = 1.0 / (denom + 1e-16)
        oh = (pv + pe * weh) * r
        head_outs.append(oh)

    if H == 1:
        out = head_outs[0]
    else:
        out = (head_outs[0] + head_outs[1]) * (1.0 / H)
    return out + skip


def _layernorm_relu(x, eps=1e-5):
    mu = jnp.mean(x, axis=-1, keepdims=True)
    d = x - mu
    var = jnp.mean(d * d, axis=-1, keepdims=True)
    return jnp.maximum(d / jnp.sqrt(var + eps), 0.0)


def _body(node_ref, edge_ref, A_ref,
          Wk0_ref, Wq0_ref, Wv0_ref, We0_ref, Ws0_ref,
          Wk1_ref, Wq1_ref, Wv1_ref, We1_ref, Ws1_ref,
          Wk2_ref, Wq2_ref, Wv2_ref, We2_ref, Ws2_ref,
          out_ref):
    b = node_ref.shape[0]
    params = [
        (jnp.concatenate([Wk0_ref[...], Wq0_ref[...], Wv0_ref[...],
                          Ws0_ref[...]], axis=1), We0_ref[...]),
        (jnp.concatenate([Wk1_ref[...], Wq1_ref[...], Wv1_ref[...],
                          Ws1_ref[...]], axis=1), We1_ref[...]),
        (jnp.concatenate([Wk2_ref[...], Wq2_ref[...], Wv2_ref[...],
                          Ws2_ref[...]], axis=1), We2_ref[...]),
    ]
    adjTs, masks, xs = [], [], []
    for j in range(b):
        adj = edge_ref[j] * A_ref[j]
        adjT = adj.T
        adjTs.append(adjT)
        masks.append(adjT != 0.0)
        xs.append(node_ref[j])

    for r in range(_RECURRENCE):
        for l in range(_DEPTH):
            if r != 0 and l == 0:
                continue
            Wall, We = params[l]
            for j in range(b):
                x = _attend(xs[j], adjTs[j], masks[j], Wall, We,
                            _H_LIST[l], first_layer=(r == 0 and l == 0))
                if l != _DEPTH - 1:
                    x = _layernorm_relu(x)
                xs[j] = x

    for j in range(b):
        out_ref[j] = xs[j]


def kernel(node, edge, A, Wk0, Wq0, Wv0, We0, Ws0,
           Wk1, Wq1, Wv1, We1, Ws1, Wk2, Wq2, Wv2, We2, Ws2):
    b, n, _ = node.shape
    return pl.pallas_call(
        _body,
        out_shape=jax.ShapeDtypeStruct((b, n, _C), jnp.float32),
    )(node, edge, A, Wk0, Wq0, Wv0, We0, Ws0,
      Wk1, Wq1, Wv1, We1, Ws1, Wk2, Wq2, Wv2, We2, Ws2)


# speedup vs baseline: 1.3976x; 1.3976x over previous
import math

import jax
import jax.numpy as jnp
from jax import lax
from jax.experimental import pallas as pl

_H_LIST = [1, 1, 2]
_C = 128
_DEPTH = 3
_RECURRENCE = 3


def _attend(x, adjT, mask, Wall, We, H, first_layer):
    if first_layer:
        proj = x * Wall
    else:
        proj = jnp.dot(x, Wall, preferred_element_type=jnp.float32)
    HC = H * _C
    k = proj[:, 0:HC]
    q = proj[:, HC:2 * HC]
    v = proj[:, 2 * HC:3 * HC]
    skip = proj[:, 3 * HC:]

    scale = 1.0 / math.sqrt(float(_C))
    head_outs = []
    for h in range(H):
        qh = q[:, h * _C:(h + 1) * _C]
        kh = k[:, h * _C:(h + 1) * _C]
        vh = v[:, h * _C:(h + 1) * _C]
        weh = We[0:1, h * _C:(h + 1) * _C]
        S = lax.dot_general(qh.astype(jnp.bfloat16), kh.astype(jnp.bfloat16),
                            (((1,), (1,)), ((), ())),
                            preferred_element_type=jnp.float32)
        qe = jnp.sum(qh * weh, axis=1, keepdims=True)
        S = S + adjT * qe
        S = jnp.where(mask, S, -jnp.inf)
        m = jnp.max(S, axis=1, keepdims=True)
        m = jnp.where(m == -jnp.inf, 0.0, m)
        P = jnp.exp((S - m) * scale)
        denom = jnp.sum(P, axis=1, keepdims=True)
        pe = jnp.sum(P * adjT, axis=1, keepdims=True)
        r = 1.0 / (denom + 1e-16)
        oh = (jnp.dot(P.astype(jnp.bfloat16), vh.astype(jnp.bfloat16),
                      preferred_element_type=jnp.float32)
              + pe * weh) * r
        head_outs.append(oh)

    if H == 1:
        out = head_outs[0]
    else:
        out = (head_outs[0] + head_outs[1]) * (1.0 / H)
    return out + skip


def _layernorm_relu(x, eps=1e-5):
    mu = jnp.mean(x, axis=-1, keepdims=True)
    d = x - mu
    var = jnp.mean(d * d, axis=-1, keepdims=True)
    return jnp.maximum(d / jnp.sqrt(var + eps), 0.0)


def _body(node_ref, edge_ref, A_ref,
          Wk0_ref, Wq0_ref, Wv0_ref, We0_ref, Ws0_ref,
          Wk1_ref, Wq1_ref, Wv1_ref, We1_ref, Ws1_ref,
          Wk2_ref, Wq2_ref, Wv2_ref, We2_ref, Ws2_ref,
          out_ref):
    b = node_ref.shape[0]
    params = [
        (jnp.concatenate([Wk0_ref[...], Wq0_ref[...], Wv0_ref[...],
                          Ws0_ref[...]], axis=1), We0_ref[...]),
        (jnp.concatenate([Wk1_ref[...], Wq1_ref[...], Wv1_ref[...],
                          Ws1_ref[...]], axis=1), We1_ref[...]),
        (jnp.concatenate([Wk2_ref[...], Wq2_ref[...], Wv2_ref[...],
                          Ws2_ref[...]], axis=1), We2_ref[...]),
    ]
    adjTs, masks, xs = [], [], []
    for j in range(b):
        adj = edge_ref[j] * A_ref[j]
        adjT = adj.T
        adjTs.append(adjT)
        masks.append(adjT != 0.0)
        xs.append(node_ref[j])

    for r in range(_RECURRENCE):
        for l in range(_DEPTH):
            if r != 0 and l == 0:
                continue
            Wall, We = params[l]
            for j in range(b):
                x = _attend(xs[j], adjTs[j], masks[j], Wall, We,
                            _H_LIST[l], first_layer=(r == 0 and l == 0))
                if l != _DEPTH - 1:
                    x = _layernorm_relu(x)
                xs[j] = x

    for j in range(b):
        out_ref[j] = xs[j]


def kernel(node, edge, A, Wk0, Wq0, Wv0, We0, Ws0,
           Wk1, Wq1, Wv1, We1, Ws1, Wk2, Wq2, Wv2, We2, Ws2):
    b, n, _ = node.shape
    return pl.pallas_call(
        _body,
        out_shape=jax.ShapeDtypeStruct((b, n, _C), jnp.float32),
    )(node, edge, A, Wk0, Wq0, Wv0, We0, Ws0,
      Wk1, Wq1, Wv1, We1, Ws1, Wk2, Wq2, Wv2, We2, Ws2)


# speedup vs baseline: 1.4673x; 1.0499x over previous
import math

import jax
import jax.numpy as jnp
from jax import lax
from jax.experimental import pallas as pl

_H_LIST = [1, 1, 2]
_C = 128
_DEPTH = 3
_RECURRENCE = 3


def _attend(x, adjT, mask, Wall, We, H, first_layer):
    if first_layer:
        proj = x * Wall
    else:
        proj = jnp.dot(x, Wall, preferred_element_type=jnp.float32)
    HC = H * _C
    k = proj[:, 0:HC]
    q = proj[:, HC:2 * HC]
    v = proj[:, 2 * HC:3 * HC]
    skip = proj[:, 3 * HC:]

    scale = 1.0 / math.sqrt(float(_C))
    head_outs = []
    for h in range(H):
        qh = q[:, h * _C:(h + 1) * _C]
        kh = k[:, h * _C:(h + 1) * _C]
        vh = v[:, h * _C:(h + 1) * _C]
        weh = We[0:1, h * _C:(h + 1) * _C]
        S = lax.dot_general(qh.astype(jnp.bfloat16), kh.astype(jnp.bfloat16),
                            (((1,), (1,)), ((), ())),
                            preferred_element_type=jnp.float32)
        qe = jnp.sum(qh * weh, axis=1, keepdims=True)
        S = S + adjT * qe
        S = jnp.where(mask, S, -jnp.inf)
        m = jnp.max(S, axis=1, keepdims=True)
        m = jnp.where(m == -jnp.inf, 0.0, m)
        P = jnp.exp((S - m) * scale)
        denom = jnp.sum(P, axis=1, keepdims=True)
        pe = jnp.sum(P * adjT, axis=1, keepdims=True)
        r = 1.0 / (denom + 1e-16)
        oh = (jnp.dot(P, vh, preferred_element_type=jnp.float32)
              + pe * weh) * r
        head_outs.append(oh)

    if H == 1:
        out = head_outs[0]
    else:
        out = (head_outs[0] + head_outs[1]) * (1.0 / H)
    return out + skip


def _layernorm_relu(x, eps=1e-5):
    mu = jnp.mean(x, axis=-1, keepdims=True)
    d = x - mu
    var = jnp.mean(d * d, axis=-1, keepdims=True)
    return jnp.maximum(d / jnp.sqrt(var + eps), 0.0)


def _body(node_ref, edge_ref, A_ref,
          Wk0_ref, Wq0_ref, Wv0_ref, We0_ref, Ws0_ref,
          Wk1_ref, Wq1_ref, Wv1_ref, We1_ref, Ws1_ref,
          Wk2_ref, Wq2_ref, Wv2_ref, We2_ref, Ws2_ref,
          out_ref):
    b = node_ref.shape[0]
    params = [
        (jnp.concatenate([Wk0_ref[...], Wq0_ref[...], Wv0_ref[...],
                          Ws0_ref[...]], axis=1), We0_ref[...]),
        (jnp.concatenate([Wk1_ref[...], Wq1_ref[...], Wv1_ref[...],
                          Ws1_ref[...]], axis=1), We1_ref[...]),
        (jnp.concatenate([Wk2_ref[...], Wq2_ref[...], Wv2_ref[...],
                          Ws2_ref[...]], axis=1), We2_ref[...]),
    ]
    adjTs, masks, xs = [], [], []
    for j in range(b):
        adj = edge_ref[j] * A_ref[j]
        adjT = adj.T
        adjTs.append(adjT)
        masks.append(adjT != 0.0)
        xs.append(node_ref[j])

    for r in range(_RECURRENCE):
        for l in range(_DEPTH):
            if r != 0 and l == 0:
                continue
            Wall, We = params[l]
            for j in range(b):
                x = _attend(xs[j], adjTs[j], masks[j], Wall, We,
                            _H_LIST[l], first_layer=(r == 0 and l == 0))
                if l != _DEPTH - 1:
                    x = _layernorm_relu(x)
                xs[j] = x

    for j in range(b):
        out_ref[j] = xs[j]


def kernel(node, edge, A, Wk0, Wq0, Wv0, We0, Ws0,
           Wk1, Wq1, Wv1, We1, Ws1, Wk2, Wq2, Wv2, We2, Ws2):
    b, n, _ = node.shape
    return pl.pallas_call(
        _body,
        out_shape=jax.ShapeDtypeStruct((b, n, _C), jnp.float32),
    )(node, edge, A, Wk0, Wq0, Wv0, We0, Ws0,
      Wk1, Wq1, Wv1, We1, Ws1, Wk2, Wq2, Wv2, We2, Ws2)


# speedup vs baseline: 1.4859x; 1.0126x over previous
import math

import jax
import jax.numpy as jnp
from jax import lax
from jax.experimental import pallas as pl

_H_LIST = [1, 1, 2]
_C = 128
_DEPTH = 3
_RECURRENCE = 3


def _attend(x, adjT, mask, Wall, We, H, first_layer):
    if first_layer:
        proj = x * Wall
    else:
        proj = jnp.dot(x, Wall, preferred_element_type=jnp.float32)
    HC = H * _C
    k = proj[:, 0:HC]
    q = proj[:, HC:2 * HC]
    v = proj[:, 2 * HC:3 * HC]
    skip = proj[:, 3 * HC:]

    scale = 1.0 / math.sqrt(float(_C))
    head_outs = []
    for h in range(H):
        qh = q[:, h * _C:(h + 1) * _C]
        kh = k[:, h * _C:(h + 1) * _C]
        vh = v[:, h * _C:(h + 1) * _C]
        weh = We[0:1, h * _C:(h + 1) * _C]
        S = lax.dot_general(qh, kh, (((1,), (1,)), ((), ())),
                            preferred_element_type=jnp.float32)
        qe = jnp.sum(qh * weh, axis=1, keepdims=True)
        S = S + adjT * qe
        S = jnp.where(mask, S, -jnp.inf)
        m = jnp.max(S, axis=1, keepdims=True)
        m = jnp.where(m == -jnp.inf, 0.0, m)
        P = jnp.exp2((S - m) * (scale * 1.4426950408889634))
        denom = jnp.sum(P, axis=1, keepdims=True)
        pe = jnp.sum(P * adjT, axis=1, keepdims=True)
        r = 1.0 / (denom + 1e-16)
        oh = (jnp.dot(P, vh, preferred_element_type=jnp.float32)
              + pe * weh) * r
        head_outs.append(oh)

    if H == 1:
        out = head_outs[0]
    else:
        out = (head_outs[0] + head_outs[1]) * (1.0 / H)
    return out + skip


def _layernorm_relu(x, eps=1e-5):
    mu = jnp.mean(x, axis=-1, keepdims=True)
    d = x - mu
    var = jnp.mean(d * d, axis=-1, keepdims=True)
    return jnp.maximum(d * lax.rsqrt(var + eps), 0.0)


def _body(node_ref, edge_ref, A_ref,
          Wk0_ref, Wq0_ref, Wv0_ref, We0_ref, Ws0_ref,
          Wk1_ref, Wq1_ref, Wv1_ref, We1_ref, Ws1_ref,
          Wk2_ref, Wq2_ref, Wv2_ref, We2_ref, Ws2_ref,
          out_ref):
    b = node_ref.shape[0]
    params = [
        (jnp.concatenate([Wk0_ref[...], Wq0_ref[...], Wv0_ref[...],
                          Ws0_ref[...]], axis=1), We0_ref[...]),
        (jnp.concatenate([Wk1_ref[...], Wq1_ref[...], Wv1_ref[...],
                          Ws1_ref[...]], axis=1), We1_ref[...]),
        (jnp.concatenate([Wk2_ref[...], Wq2_ref[...], Wv2_ref[...],
                          Ws2_ref[...]], axis=1), We2_ref[...]),
    ]
    adjTs, masks, xs = [], [], []
    for j in range(b):
        adj = edge_ref[j] * A_ref[j]
        adjT = adj.T
        adjTs.append(adjT)
        masks.append(adjT != 0.0)
        xs.append(node_ref[j])

    for r in range(_RECURRENCE):
        for l in range(_DEPTH):
            if r != 0 and l == 0:
                continue
            Wall, We = params[l]
            for j in range(b):
                x = _attend(xs[j], adjTs[j], masks[j], Wall, We,
                            _H_LIST[l], first_layer=(r == 0 and l == 0))
                if l != _DEPTH - 1:
                    x = _layernorm_relu(x)
                xs[j] = x

    for j in range(b):
        out_ref[j] = xs[j]


def kernel(node, edge, A, Wk0, Wq0, Wv0, We0, Ws0,
           Wk1, Wq1, Wv1, We1, Ws1, Wk2, Wq2, Wv2, We2, Ws2):
    b, n, _ = node.shape
    return pl.pallas_call(
        _body,
        out_shape=jax.ShapeDtypeStruct((b, n, _C), jnp.float32),
    )(node, edge, A, Wk0, Wq0, Wv0, We0, Ws0,
      Wk1, Wq1, Wv1, We1, Ws1, Wk2, Wq2, Wv2, We2, Ws2)
